# Initial kernel scaffold; baseline (speedup 1.0000x reference)
import jax
import jax.numpy as jnp
from jax.experimental import pallas as pl


def kernel(node_type, in_degree, out_degree, node_weight, in_degree_weight, out_degree_weight):
    raise NotImplementedError("write your pallas kernel here")


# speedup vs baseline: 1.6713x; 1.6713x over previous
import functools

import jax
import jax.numpy as jnp
from jax import lax
from jax.experimental import pallas as pl
from jax.experimental.pallas import tpu as pltpu
from jax.experimental.pallas import tpu_sc as plsc

N_NODES = 100000
N_FEATS = 8
D = 128
C2 = D // 2
K = N_FEATS + 2
VOCAB = 513 + 256 + 256

NC = 2
NS = 16
NW = NC * NS
PER_W = 3200
N_PAD = NW * PER_W
GROUPS = PER_W // 16
OUT_GROUPS = 4


def _sc_body(tbl_hbm, idx_hbm, out_hbm, tbl_v, idx_v, out_v):
  c = lax.axis_index("c")
  s = lax.axis_index("s")
  w = s * NC + c
  base = w * PER_W

  pltpu.sync_copy(tbl_hbm, tbl_v)
  pltpu.sync_copy(idx_hbm.at[:, pl.ds(base * 1, PER_W)], idx_v)

  iota = lax.iota(jnp.int32, 16)
  row_off = iota * D

  def grp(g, carry):
    gi = lax.rem(g, OUT_GROUPS)
    idxs = [idx_v[j, pl.ds(g * 16, 16)] for j in range(K)]
    sbase = gi * (16 * D) + row_off
    for c2 in range(C2):
      acc_a = None
      acc_b = None
      for j in range(K):
        word = plsc.load_gather(tbl_v, [idxs[j] + c2])
        ab = plsc.bitcast(word, jnp.bfloat16)
        a, b = plsc.unpack(ab, format=plsc.PackFormat.INTERLEAVED)
        acc_a = a if acc_a is None else acc_a + a
        acc_b = b if acc_b is None else acc_b + b
      plsc.store_scatter(out_v, [sbase + (2 * c2)], acc_a)
      plsc.store_scatter(out_v, [sbase + (2 * c2 + 1)], acc_b)

    @pl.when(gi == OUT_GROUPS - 1)
    def _flush():
      row0 = base + (g - (OUT_GROUPS - 1)) * 16
      pltpu.sync_copy(out_v, out_hbm.at[pl.ds(row0 * D, OUT_GROUPS * 16 * D)])

    return carry

  lax.fori_loop(0, GROUPS, grp, 0)


@jax.jit
def _run(tbl_packed, idx10):
  mesh = plsc.VectorSubcoreMesh(
      core_axis_name="c", subcore_axis_name="s", num_cores=NC, num_subcores=NS
  )
  f = pl.kernel(
      _sc_body,
      out_type=jax.ShapeDtypeStruct((N_PAD * D,), jnp.float32),
      mesh=mesh,
      scratch_types=[
          pltpu.VMEM((VOCAB * C2,), jnp.int32),
          pltpu.VMEM((K, PER_W), jnp.int32),
          pltpu.VMEM((OUT_GROUPS * 16 * D,), jnp.float32),
      ],
      compiler_params=pltpu.CompilerParams(needs_layout_passes=False),
  )
  return f(tbl_packed, idx10)


def kernel(node_type, in_degree, out_degree, node_weight, in_degree_weight,
           out_degree_weight):
  big = jnp.concatenate(
      [node_weight, in_degree_weight, out_degree_weight], axis=0
  ).astype(jnp.bfloat16)
  tbl_packed = lax.bitcast_convert_type(
      big.reshape(VOCAB, C2, 2), jnp.int32
  ).reshape(VOCAB * C2)

  nt = node_type.astype(jnp.int32).T
  ind = in_degree.astype(jnp.int32)[None, :] + 513
  outd = out_degree.astype(jnp.int32)[None, :] + (513 + 256)
  idx10 = jnp.concatenate([nt, ind, outd], axis=0) * C2
  idx10 = jnp.pad(idx10, ((0, 0), (0, N_PAD - N_NODES)))

  out = _run(tbl_packed, idx10)
  return out.reshape(N_PAD, D)[:N_NODES]
